# baseline (speedup 1.0000x reference)
import jax
import jax.numpy as jnp
import numpy as _np
from jax.experimental import pallas as pl
from jax.experimental.pallas import tpu as pltpu

B, N, G, C = 8, 20000, 64, 80
FG_THRESH, BG_THRESH = 0.5, 0.4
ALPHA, GAMMA = 0.25, 2.0
EPS = 1e-7
DW_CLAMP = float(_np.log(1000.0 / 16.0))
PI = float(_np.pi)

T = 2048


def _atan_pos(x):
    big = x > 1.0
    y = jnp.where(big, 1.0 / x, x)
    y = y / (1.0 + jnp.sqrt(1.0 + y * y))
    y = y / (1.0 + jnp.sqrt(1.0 + y * y))
    t2 = y * y
    at = y * (1.0 + t2 * (-1.0 / 3.0 + t2 * (1.0 / 5.0 + t2 * (-1.0 / 7.0 + t2 * (1.0 / 9.0)))))
    at = 4.0 * at
    return jnp.where(big, PI / 2.0 - at, at)


def _kernel(pbt_ref, plabt_ref, anct_ref, tbc_ref, tb5_ref, acc_ref,
            hi_ref, iou_ref):
    ph = pl.program_id(1)
    t = pl.program_id(2)

    col = t * T + jax.lax.broadcasted_iota(jnp.int32, (1, T), 1)
    padm = col < N

    @pl.when(ph == 0)
    def _():
        anc = jnp.transpose(anct_ref[0])
        tbc = tbc_ref[0]
        ax0, ay0 = anc[0:1, :], anc[1:2, :]
        ax1, ay1 = anc[2:3, :], anc[3:4, :]
        bx0, by0 = tbc[:, 0:1], tbc[:, 1:2]
        bx1, by1 = tbc[:, 2:3], tbc[:, 3:4]
        area_a = (ax1 - ax0) * (ay1 - ay0)
        area_b = (bx1 - bx0) * (by1 - by0)
        w = jnp.maximum(jnp.minimum(bx1, ax1) - jnp.maximum(bx0, ax0), 0.0)
        h = jnp.maximum(jnp.minimum(by1, ay1) - jnp.maximum(by0, ay0), 0.0)
        inter = w * h
        iou = inter / (area_b + area_a - inter + EPS)
        iou = jnp.where(padm, iou, -1.0)
        iou_ref[:, pl.ds(t * T, T)] = iou
        part = jnp.max(iou, axis=1, keepdims=True)
        hi0 = jnp.where(t == 0, jnp.full((G, 1), -jnp.inf, jnp.float32),
                        hi_ref[...])
        hi_ref[...] = jnp.maximum(hi0, part)

    @pl.when(ph == 1)
    def _():
        iou = iou_ref[:, pl.ds(t * T, T)]
        mv = jnp.max(iou, axis=0, keepdims=True)
        gidx = jax.lax.broadcasted_iota(jnp.int32, (G, T), 0)
        am = jnp.min(jnp.where(iou == mv, gidx, G), axis=0, keepdims=True)
        matches = jnp.where(mv < BG_THRESH, -1, am)
        matches = jnp.where((mv >= BG_THRESH) & (mv < FG_THRESH), -2,
                            matches)
        hi = hi_ref[...]
        force = jnp.max(jnp.where(iou == hi, 1.0, 0.0), axis=0,
                        keepdims=True) > 0.0
        matches = jnp.where(force, am, matches)
        fgb = (matches >= 0) & padm
        validb = (matches != -2) & padm
        midx = jnp.maximum(matches, 0)
        onehot = (gidx == midx).astype(jnp.float32)

        g5 = jax.lax.dot_general(tb5_ref[0], onehot,
                                 (((1,), (0,)), ((), ())),
                                 preferred_element_type=jnp.float32)
        gx0, gy0 = g5[0:1, :], g5[1:2, :]
        gx1, gy1 = g5[2:3, :], g5[3:4, :]
        lab = g5[4:5, :].astype(jnp.int32)

        x = jnp.transpose(plabt_ref[0])
        e = jnp.exp(-jnp.abs(x))
        lg = jnp.log1p(e)
        r = 1.0 / (1.0 + e)
        xpos = x >= 0.0
        sp = jnp.where(xpos, x, 0.0) + lg
        p = jnp.where(xpos, r, e * r)
        base = jnp.sum((1.0 - ALPHA) * sp * p * p, axis=0, keepdims=True)
        cidx = jax.lax.broadcasted_iota(jnp.int32, (C, T), 0)
        xsel = jnp.sum(jnp.where(cidx == lab, x, 0.0), axis=0,
                       keepdims=True)
        e1 = jnp.exp(-jnp.abs(xsel))
        lg1 = jnp.log1p(e1)
        r1 = 1.0 / (1.0 + e1)
        xp1 = xsel >= 0.0
        sp1 = jnp.where(xp1, xsel, 0.0) + lg1
        sn1 = jnp.where(xp1, 0.0, -xsel) + lg1
        p1 = jnp.where(xp1, r1, e1 * r1)
        q1 = 1.0 - p1
        corr = ALPHA * sn1 * q1 * q1 - (1.0 - ALPHA) * sp1 * p1 * p1
        cls_sum = jnp.sum(jnp.where(validb, base, 0.0) +
                          jnp.where(fgb, corr, 0.0))

        anc = jnp.transpose(anct_ref[0])
        ax0, ay0 = anc[0:1, :], anc[1:2, :]
        ax1, ay1 = anc[2:3, :], anc[3:4, :]
        aw = ax1 - ax0
        ah = ay1 - ay0
        cx = ax0 + 0.5 * aw
        cy = ay0 + 0.5 * ah
        pbt = jnp.transpose(pbt_ref[0])
        dx, dy = pbt[0:1, :], pbt[1:2, :]
        dw = jnp.minimum(pbt[2:3, :], DW_CLAMP)
        dh = jnp.minimum(pbt[3:4, :], DW_CLAMP)
        pcx = dx * aw + cx
        pcy = dy * ah + cy
        pw = jnp.exp(dw) * aw
        phh0 = jnp.exp(dh) * ah
        px0 = pcx - 0.5 * pw
        py0 = pcy - 0.5 * phh0
        px1 = pcx + 0.5 * pw
        py1 = pcy + 0.5 * phh0

        iw = jnp.maximum(jnp.minimum(px1, gx1) - jnp.maximum(px0, gx0), 0.0)
        ih = jnp.maximum(jnp.minimum(py1, gy1) - jnp.maximum(py0, gy0), 0.0)
        binter = iw * ih
        pww = px1 - px0
        phh = py1 - py0
        gww = gx1 - gx0
        ghh = gy1 - gy0
        union = pww * phh + gww * ghh - binter
        biou = binter / (union + EPS)
        cw = jnp.maximum(px1, gx1) - jnp.minimum(px0, gx0)
        ch = jnp.maximum(py1, gy1) - jnp.minimum(py0, gy0)
        c2 = cw * cw + ch * ch + EPS
        rho2 = ((px0 + px1) * 0.5 - (gx0 + gx1) * 0.5) ** 2 + \
               ((py0 + py1) * 0.5 - (gy0 + gy1) * 0.5) ** 2
        v = (4.0 / (PI * PI)) * (_atan_pos(gww / (ghh + EPS)) -
                                 _atan_pos(pww / (phh + EPS))) ** 2
        alpha_t = v / (1.0 - biou + v + EPS)
        bl = 1.0 - (biou - rho2 / c2 - alpha_t * v)
        box_sum = jnp.sum(jnp.where(fgb, bl, 0.0))
        nfg = jnp.sum(jnp.where(fgb, 1.0, 0.0))

        lane = jax.lax.broadcasted_iota(jnp.int32, (1, 128), 1)
        row = jnp.where(lane == 0, cls_sum, 0.0) + \
              jnp.where(lane == 1, box_sum, 0.0) + \
              jnp.where(lane == 2, nfg, 0.0)

        @pl.when(t == 0)
        def _():
            acc_ref[0] = jnp.zeros((1, 128), jnp.float32)

        acc_ref[0] = acc_ref[0] + row


@jax.jit
def kernel(pred_boxes, pred_labels, target_boxes, target_labels, anchors):
    tlf = target_labels.astype(jnp.float32)[:, :, None]
    tb5 = jnp.concatenate(
        [jnp.transpose(target_boxes, (0, 2, 1)), tlf.transpose(0, 2, 1)],
        axis=1)

    nt = pl.cdiv(N, T)
    acc = pl.pallas_call(
        _kernel,
        grid=(B, 2, nt),
        in_specs=[
            pl.BlockSpec((1, T, 4), lambda b, ph, t: (b, t * ph, 0)),
            pl.BlockSpec((1, T, C), lambda b, ph, t: (b, t * ph, 0)),
            pl.BlockSpec((1, T, 4), lambda b, ph, t: (b, t, 0)),
            pl.BlockSpec((1, G, 4), lambda b, ph, t: (b, 0, 0)),
            pl.BlockSpec((1, 5, G), lambda b, ph, t: (b, 0, 0)),
        ],
        out_specs=pl.BlockSpec((1, 1, 128), lambda b, ph, t: (b, 0, 0)),
        out_shape=jax.ShapeDtypeStruct((B, 1, 128), jnp.float32),
        scratch_shapes=[pltpu.VMEM((G, 1), jnp.float32),
                        pltpu.VMEM((G, T * ((N + T - 1) // T)), jnp.float32)],
    )(pred_boxes, pred_labels, anchors, target_boxes, tb5)

    cls_sum = acc[:, 0, 0]
    box_sum = acc[:, 0, 1]
    nfg = acc[:, 0, 2]
    denom = jnp.maximum(1.0, nfg)
    cls = cls_sum / denom
    box = box_sum / denom
    return jnp.stack([cls.mean(), box.mean()])


# speedup vs baseline: 2.1153x; 2.1153x over previous
import jax
import jax.numpy as jnp
import numpy as _np
from jax.experimental import pallas as pl
from jax.experimental.pallas import tpu as pltpu

B, N, G, C = 8, 20000, 64, 80
FG_THRESH, BG_THRESH = 0.5, 0.4
ALPHA, GAMMA = 0.25, 2.0
EPS = 1e-7
DW_CLAMP = float(_np.log(1000.0 / 16.0))
PI = float(_np.pi)

T = 2048


def _atan_pos(x):
    big = x > 1.0
    y = jnp.where(big, 1.0 / x, x)
    y = y / (1.0 + jnp.sqrt(1.0 + y * y))
    y = y / (1.0 + jnp.sqrt(1.0 + y * y))
    t2 = y * y
    at = y * (1.0 + t2 * (-1.0 / 3.0 + t2 * (1.0 / 5.0 + t2 * (-1.0 / 7.0 + t2 * (1.0 / 9.0)))))
    at = 4.0 * at
    return jnp.where(big, PI / 2.0 - at, at)


def _kernel(pbt_ref, plabt_ref, anct_ref, tbc_ref, tb5_ref, acc_ref,
            hi_ref, iou_ref):
    ph = pl.program_id(1)
    t = pl.program_id(2)

    col = t * T + jax.lax.broadcasted_iota(jnp.int32, (1, T), 1)
    padm = col < N

    @pl.when(ph == 0)
    def _():
        anc = anct_ref[0]
        tbc = tbc_ref[0]
        ax0, ay0 = anc[0:1, :], anc[1:2, :]
        ax1, ay1 = anc[2:3, :], anc[3:4, :]
        bx0, by0 = tbc[:, 0:1], tbc[:, 1:2]
        bx1, by1 = tbc[:, 2:3], tbc[:, 3:4]
        area_a = (ax1 - ax0) * (ay1 - ay0)
        area_b = (bx1 - bx0) * (by1 - by0)
        w = jnp.maximum(jnp.minimum(bx1, ax1) - jnp.maximum(bx0, ax0), 0.0)
        h = jnp.maximum(jnp.minimum(by1, ay1) - jnp.maximum(by0, ay0), 0.0)
        inter = w * h
        iou = inter / (area_b + area_a - inter + EPS)
        iou = jnp.where(padm, iou, -1.0)
        iou_ref[:, pl.ds(t * T, T)] = iou
        part = jnp.max(iou, axis=1, keepdims=True)
        hi0 = jnp.where(t == 0, jnp.full((G, 1), -jnp.inf, jnp.float32),
                        hi_ref[...])
        hi_ref[...] = jnp.maximum(hi0, part)

    @pl.when(ph == 1)
    def _():
        iou = iou_ref[:, pl.ds(t * T, T)]
        mv = jnp.max(iou, axis=0, keepdims=True)
        gidx = jax.lax.broadcasted_iota(jnp.int32, (G, T), 0)
        am = jnp.min(jnp.where(iou == mv, gidx, G), axis=0, keepdims=True)
        matches = jnp.where(mv < BG_THRESH, -1, am)
        matches = jnp.where((mv >= BG_THRESH) & (mv < FG_THRESH), -2,
                            matches)
        hi = hi_ref[...]
        force = jnp.max(jnp.where(iou == hi, 1.0, 0.0), axis=0,
                        keepdims=True) > 0.0
        matches = jnp.where(force, am, matches)
        fgb = (matches >= 0) & padm
        validb = (matches != -2) & padm
        midx = jnp.maximum(matches, 0)
        onehot = (gidx == midx).astype(jnp.float32)

        g5 = jax.lax.dot_general(tb5_ref[0], onehot,
                                 (((1,), (0,)), ((), ())),
                                 preferred_element_type=jnp.float32)
        gx0, gy0 = g5[0:1, :], g5[1:2, :]
        gx1, gy1 = g5[2:3, :], g5[3:4, :]
        lab = g5[4:5, :].astype(jnp.int32)

        x = plabt_ref[0]
        e = jnp.exp(-jnp.abs(x))
        lg = jnp.log1p(e)
        r = 1.0 / (1.0 + e)
        xpos = x >= 0.0
        sp = jnp.where(xpos, x, 0.0) + lg
        p = jnp.where(xpos, r, e * r)
        negterm = sp * (p * p)
        ones_c = jnp.ones((1, C), jnp.float32)
        base = (1.0 - ALPHA) * jax.lax.dot_general(
            ones_c, negterm, (((1,), (0,)), ((), ())),
            preferred_element_type=jnp.float32)
        cidx = jax.lax.broadcasted_iota(jnp.int32, (C, T), 0)
        xsel = jnp.sum(jnp.where(cidx == lab, x, 0.0), axis=0,
                       keepdims=True)
        e1 = jnp.exp(-jnp.abs(xsel))
        lg1 = jnp.log1p(e1)
        r1 = 1.0 / (1.0 + e1)
        xp1 = xsel >= 0.0
        sp1 = jnp.where(xp1, xsel, 0.0) + lg1
        sn1 = jnp.where(xp1, 0.0, -xsel) + lg1
        p1 = jnp.where(xp1, r1, e1 * r1)
        q1 = 1.0 - p1
        corr = ALPHA * sn1 * q1 * q1 - (1.0 - ALPHA) * sp1 * p1 * p1
        cls_sum = jnp.sum(jnp.where(validb, base, 0.0) +
                          jnp.where(fgb, corr, 0.0))

        anc = anct_ref[0]
        ax0, ay0 = anc[0:1, :], anc[1:2, :]
        ax1, ay1 = anc[2:3, :], anc[3:4, :]
        aw = ax1 - ax0
        ah = ay1 - ay0
        cx = ax0 + 0.5 * aw
        cy = ay0 + 0.5 * ah
        pbt = pbt_ref[0]
        dx, dy = pbt[0:1, :], pbt[1:2, :]
        dw = jnp.minimum(pbt[2:3, :], DW_CLAMP)
        dh = jnp.minimum(pbt[3:4, :], DW_CLAMP)
        pcx = dx * aw + cx
        pcy = dy * ah + cy
        pw = jnp.exp(dw) * aw
        phh0 = jnp.exp(dh) * ah
        px0 = pcx - 0.5 * pw
        py0 = pcy - 0.5 * phh0
        px1 = pcx + 0.5 * pw
        py1 = pcy + 0.5 * phh0

        iw = jnp.maximum(jnp.minimum(px1, gx1) - jnp.maximum(px0, gx0), 0.0)
        ih = jnp.maximum(jnp.minimum(py1, gy1) - jnp.maximum(py0, gy0), 0.0)
        binter = iw * ih
        pww = px1 - px0
        phh = py1 - py0
        gww = gx1 - gx0
        ghh = gy1 - gy0
        union = pww * phh + gww * ghh - binter
        biou = binter / (union + EPS)
        cw = jnp.maximum(px1, gx1) - jnp.minimum(px0, gx0)
        ch = jnp.maximum(py1, gy1) - jnp.minimum(py0, gy0)
        c2 = cw * cw + ch * ch + EPS
        rho2 = ((px0 + px1) * 0.5 - (gx0 + gx1) * 0.5) ** 2 + \
               ((py0 + py1) * 0.5 - (gy0 + gy1) * 0.5) ** 2
        v = (4.0 / (PI * PI)) * (_atan_pos(gww / (ghh + EPS)) -
                                 _atan_pos(pww / (phh + EPS))) ** 2
        alpha_t = v / (1.0 - biou + v + EPS)
        bl = 1.0 - (biou - rho2 / c2 - alpha_t * v)
        box_sum = jnp.sum(jnp.where(fgb, bl, 0.0))
        nfg = jnp.sum(jnp.where(fgb, 1.0, 0.0))

        lane = jax.lax.broadcasted_iota(jnp.int32, (1, 128), 1)
        row = jnp.where(lane == 0, cls_sum, 0.0) + \
              jnp.where(lane == 1, box_sum, 0.0) + \
              jnp.where(lane == 2, nfg, 0.0)

        @pl.when(t == 0)
        def _():
            acc_ref[0] = jnp.zeros((1, 128), jnp.float32)

        acc_ref[0] = acc_ref[0] + row


@jax.jit
def kernel(pred_boxes, pred_labels, target_boxes, target_labels, anchors):
    anct = jnp.transpose(anchors, (0, 2, 1))
    pbt = jnp.transpose(pred_boxes, (0, 2, 1))
    plabt = jnp.transpose(pred_labels, (0, 2, 1))
    tlf = target_labels.astype(jnp.float32)[:, :, None]
    tb5 = jnp.concatenate(
        [jnp.transpose(target_boxes, (0, 2, 1)), tlf.transpose(0, 2, 1)],
        axis=1)

    nt = pl.cdiv(N, T)
    acc = pl.pallas_call(
        _kernel,
        grid=(B, 2, nt),
        in_specs=[
            pl.BlockSpec((1, 4, T), lambda b, ph, t: (b, 0, t * ph)),
            pl.BlockSpec((1, C, T), lambda b, ph, t: (b, 0, t * ph)),
            pl.BlockSpec((1, 4, T), lambda b, ph, t: (b, 0, t)),
            pl.BlockSpec((1, G, 4), lambda b, ph, t: (b, 0, 0)),
            pl.BlockSpec((1, 5, G), lambda b, ph, t: (b, 0, 0)),
        ],
        out_specs=pl.BlockSpec((1, 1, 128), lambda b, ph, t: (b, 0, 0)),
        out_shape=jax.ShapeDtypeStruct((B, 1, 128), jnp.float32),
        scratch_shapes=[pltpu.VMEM((G, 1), jnp.float32),
                        pltpu.VMEM((G, T * ((N + T - 1) // T)), jnp.float32)],
    )(pbt, plabt, anct, target_boxes, tb5)

    cls_sum = acc[:, 0, 0]
    box_sum = acc[:, 0, 1]
    nfg = acc[:, 0, 2]
    denom = jnp.maximum(1.0, nfg)
    cls = cls_sum / denom
    box = box_sum / denom
    return jnp.stack([cls.mean(), box.mean()])


# speedup vs baseline: 2.1813x; 1.0312x over previous
import jax
import jax.numpy as jnp
import numpy as _np
from jax.experimental import pallas as pl
from jax.experimental.pallas import tpu as pltpu

B, N, G, C = 8, 20000, 64, 80
FG_THRESH, BG_THRESH = 0.5, 0.4
ALPHA, GAMMA = 0.25, 2.0
EPS = 1e-7
DW_CLAMP = float(_np.log(1000.0 / 16.0))
PI = float(_np.pi)

T = 2048


def _atan_pos(x):
    big = x > 1.0
    y = jnp.where(big, 1.0 / x, x)
    y = y / (1.0 + jnp.sqrt(1.0 + y * y))
    y = y / (1.0 + jnp.sqrt(1.0 + y * y))
    t2 = y * y
    at = y * (1.0 + t2 * (-1.0 / 3.0 + t2 * (1.0 / 5.0 + t2 * (-1.0 / 7.0 + t2 * (1.0 / 9.0)))))
    at = 4.0 * at
    return jnp.where(big, PI / 2.0 - at, at)


def _kernel(pbt_ref, plabt_ref, anct_ref, tbc_ref, tb5_ref, acc_ref,
            hi_ref, iou_ref):
    ph = pl.program_id(1)
    t = pl.program_id(2)

    col = t * T + jax.lax.broadcasted_iota(jnp.int32, (1, T), 1)
    padm = col < N

    @pl.when(ph == 0)
    def _():
        anc = anct_ref[0]
        tbc = tbc_ref[0]
        ax0, ay0 = anc[0:1, :], anc[1:2, :]
        ax1, ay1 = anc[2:3, :], anc[3:4, :]
        bx0, by0 = tbc[:, 0:1], tbc[:, 1:2]
        bx1, by1 = tbc[:, 2:3], tbc[:, 3:4]
        area_a = (ax1 - ax0) * (ay1 - ay0)
        area_b = (bx1 - bx0) * (by1 - by0)
        w = jnp.maximum(jnp.minimum(bx1, ax1) - jnp.maximum(bx0, ax0), 0.0)
        h = jnp.maximum(jnp.minimum(by1, ay1) - jnp.maximum(by0, ay0), 0.0)
        inter = w * h
        iou = inter / (area_b + area_a - inter + EPS)
        iou = jnp.where(padm, iou, -1.0)
        iou_ref[:, pl.ds(t * T, T)] = iou
        part = jnp.max(iou, axis=1, keepdims=True)
        hi0 = jnp.where(t == 0, jnp.full((G, 1), -jnp.inf, jnp.float32),
                        hi_ref[...])
        hi_ref[...] = jnp.maximum(hi0, part)

    @pl.when(ph == 1)
    def _():
        iou = iou_ref[:, pl.ds(t * T, T)]
        mv = jnp.max(iou, axis=0, keepdims=True)
        gidx = jax.lax.broadcasted_iota(jnp.int32, (G, T), 0)
        am = jnp.min(jnp.where(iou == mv, gidx, G), axis=0, keepdims=True)
        matches = jnp.where(mv < BG_THRESH, -1, am)
        matches = jnp.where((mv >= BG_THRESH) & (mv < FG_THRESH), -2,
                            matches)
        hi = hi_ref[...]
        ones_g = jnp.ones((1, G), jnp.float32)
        force = jax.lax.dot_general(
            ones_g, jnp.where(iou == hi, 1.0, 0.0),
            (((1,), (0,)), ((), ())),
            preferred_element_type=jnp.float32) > 0.5
        matches = jnp.where(force, am, matches)
        fgb = (matches >= 0) & padm
        validb = (matches != -2) & padm
        midx = jnp.maximum(matches, 0)
        onehot = (gidx == midx).astype(jnp.float32)

        g5 = jax.lax.dot_general(tb5_ref[0], onehot,
                                 (((1,), (0,)), ((), ())),
                                 preferred_element_type=jnp.float32)
        gx0, gy0 = g5[0:1, :], g5[1:2, :]
        gx1, gy1 = g5[2:3, :], g5[3:4, :]
        lab = g5[4:5, :].astype(jnp.int32)

        x = plabt_ref[0]
        e = jnp.exp(-jnp.abs(x))
        lg = jnp.log1p(e)
        r = 1.0 / (1.0 + e)
        xpos = x >= 0.0
        sp = jnp.where(xpos, x, 0.0) + lg
        p = jnp.where(xpos, r, e * r)
        negterm = sp * (p * p)
        ones_c = jnp.ones((1, C), jnp.float32)
        base = (1.0 - ALPHA) * jax.lax.dot_general(
            ones_c, negterm, (((1,), (0,)), ((), ())),
            preferred_element_type=jnp.float32)
        cidx = jax.lax.broadcasted_iota(jnp.int32, (C, T), 0)
        xsel = jax.lax.dot_general(
            ones_c, jnp.where(cidx == lab, x, 0.0),
            (((1,), (0,)), ((), ())),
            preferred_element_type=jnp.float32)
        e1 = jnp.exp(-jnp.abs(xsel))
        lg1 = jnp.log1p(e1)
        r1 = 1.0 / (1.0 + e1)
        xp1 = xsel >= 0.0
        sp1 = jnp.where(xp1, xsel, 0.0) + lg1
        sn1 = jnp.where(xp1, 0.0, -xsel) + lg1
        p1 = jnp.where(xp1, r1, e1 * r1)
        q1 = 1.0 - p1
        corr = ALPHA * sn1 * q1 * q1 - (1.0 - ALPHA) * sp1 * p1 * p1
        cls_sum = jnp.sum(jnp.where(validb, base, 0.0) +
                          jnp.where(fgb, corr, 0.0))

        anc = anct_ref[0]
        ax0, ay0 = anc[0:1, :], anc[1:2, :]
        ax1, ay1 = anc[2:3, :], anc[3:4, :]
        aw = ax1 - ax0
        ah = ay1 - ay0
        cx = ax0 + 0.5 * aw
        cy = ay0 + 0.5 * ah
        pbt = pbt_ref[0]
        dx, dy = pbt[0:1, :], pbt[1:2, :]
        dw = jnp.minimum(pbt[2:3, :], DW_CLAMP)
        dh = jnp.minimum(pbt[3:4, :], DW_CLAMP)
        pcx = dx * aw + cx
        pcy = dy * ah + cy
        pw = jnp.exp(dw) * aw
        phh0 = jnp.exp(dh) * ah
        px0 = pcx - 0.5 * pw
        py0 = pcy - 0.5 * phh0
        px1 = pcx + 0.5 * pw
        py1 = pcy + 0.5 * phh0

        iw = jnp.maximum(jnp.minimum(px1, gx1) - jnp.maximum(px0, gx0), 0.0)
        ih = jnp.maximum(jnp.minimum(py1, gy1) - jnp.maximum(py0, gy0), 0.0)
        binter = iw * ih
        pww = px1 - px0
        phh = py1 - py0
        gww = gx1 - gx0
        ghh = gy1 - gy0
        union = pww * phh + gww * ghh - binter
        biou = binter / (union + EPS)
        cw = jnp.maximum(px1, gx1) - jnp.minimum(px0, gx0)
        ch = jnp.maximum(py1, gy1) - jnp.minimum(py0, gy0)
        c2 = cw * cw + ch * ch + EPS
        rho2 = ((px0 + px1) * 0.5 - (gx0 + gx1) * 0.5) ** 2 + \
               ((py0 + py1) * 0.5 - (gy0 + gy1) * 0.5) ** 2
        v = (4.0 / (PI * PI)) * (_atan_pos(gww / (ghh + EPS)) -
                                 _atan_pos(pww / (phh + EPS))) ** 2
        alpha_t = v / (1.0 - biou + v + EPS)
        bl = 1.0 - (biou - rho2 / c2 - alpha_t * v)
        box_sum = jnp.sum(jnp.where(fgb, bl, 0.0))
        nfg = jnp.sum(jnp.where(fgb, 1.0, 0.0))

        lane = jax.lax.broadcasted_iota(jnp.int32, (1, 128), 1)
        row = jnp.where(lane == 0, cls_sum, 0.0) + \
              jnp.where(lane == 1, box_sum, 0.0) + \
              jnp.where(lane == 2, nfg, 0.0)

        @pl.when(t == 0)
        def _():
            acc_ref[0] = jnp.zeros((1, 128), jnp.float32)

        acc_ref[0] = acc_ref[0] + row


@jax.jit
def kernel(pred_boxes, pred_labels, target_boxes, target_labels, anchors):
    anct = jnp.transpose(anchors, (0, 2, 1))
    pbt = jnp.transpose(pred_boxes, (0, 2, 1))
    plabt = jnp.transpose(pred_labels, (0, 2, 1))
    tlf = target_labels.astype(jnp.float32)[:, :, None]
    tb5 = jnp.concatenate(
        [jnp.transpose(target_boxes, (0, 2, 1)), tlf.transpose(0, 2, 1)],
        axis=1)

    nt = pl.cdiv(N, T)
    acc = pl.pallas_call(
        _kernel,
        grid=(B, 2, nt),
        in_specs=[
            pl.BlockSpec((1, 4, T), lambda b, ph, t: (b, 0, t * ph)),
            pl.BlockSpec((1, C, T), lambda b, ph, t: (b, 0, t * ph)),
            pl.BlockSpec((1, 4, T), lambda b, ph, t: (b, 0, t)),
            pl.BlockSpec((1, G, 4), lambda b, ph, t: (b, 0, 0)),
            pl.BlockSpec((1, 5, G), lambda b, ph, t: (b, 0, 0)),
        ],
        out_specs=pl.BlockSpec((1, 1, 128), lambda b, ph, t: (b, 0, 0)),
        out_shape=jax.ShapeDtypeStruct((B, 1, 128), jnp.float32),
        scratch_shapes=[pltpu.VMEM((G, 1), jnp.float32),
                        pltpu.VMEM((G, T * ((N + T - 1) // T)), jnp.float32)],
    )(pbt, plabt, anct, target_boxes, tb5)

    cls_sum = acc[:, 0, 0]
    box_sum = acc[:, 0, 1]
    nfg = acc[:, 0, 2]
    denom = jnp.maximum(1.0, nfg)
    cls = cls_sum / denom
    box = box_sum / denom
    return jnp.stack([cls.mean(), box.mean()])


# speedup vs baseline: 2.6511x; 1.2154x over previous
import jax
import jax.numpy as jnp
import numpy as _np
from jax.experimental import pallas as pl
from jax.experimental.pallas import tpu as pltpu

B, N, G, C = 8, 20000, 64, 80
FG_THRESH, BG_THRESH = 0.5, 0.4
ALPHA, GAMMA = 0.25, 2.0
EPS = 1e-7
DW_CLAMP = float(_np.log(1000.0 / 16.0))
PI = float(_np.pi)

T = 4096


def _atan_pos(x):
    big = x > 1.0
    y = jnp.where(big, 1.0 / x, x)
    y = y / (1.0 + jnp.sqrt(1.0 + y * y))
    y = y / (1.0 + jnp.sqrt(1.0 + y * y))
    t2 = y * y
    at = y * (1.0 + t2 * (-1.0 / 3.0 + t2 * (1.0 / 5.0 + t2 * (-1.0 / 7.0 + t2 * (1.0 / 9.0)))))
    at = 4.0 * at
    return jnp.where(big, PI / 2.0 - at, at)


def _kernel(pbt_ref, plabt_ref, anct_ref, tbc_ref, tb5_ref, acc_ref,
            hi_ref, iou_ref):
    ph = pl.program_id(1)
    t = pl.program_id(2)

    col = t * T + jax.lax.broadcasted_iota(jnp.int32, (1, T), 1)
    padm = col < N

    @pl.when(ph == 0)
    def _():
        anc = anct_ref[0]
        tbc = tbc_ref[0]
        ax0, ay0 = anc[0:1, :], anc[1:2, :]
        ax1, ay1 = anc[2:3, :], anc[3:4, :]
        bx0, by0 = tbc[:, 0:1], tbc[:, 1:2]
        bx1, by1 = tbc[:, 2:3], tbc[:, 3:4]
        area_a = (ax1 - ax0) * (ay1 - ay0)
        area_b = (bx1 - bx0) * (by1 - by0)
        w = jnp.maximum(jnp.minimum(bx1, ax1) - jnp.maximum(bx0, ax0), 0.0)
        h = jnp.maximum(jnp.minimum(by1, ay1) - jnp.maximum(by0, ay0), 0.0)
        inter = w * h
        iou = inter / (area_b + area_a - inter + EPS)
        iou = jnp.where(padm, iou, -1.0)
        iou_ref[:, pl.ds(t * T, T)] = iou
        part = jnp.max(iou, axis=1, keepdims=True)
        hi0 = jnp.where(t == 0, jnp.full((G, 1), -jnp.inf, jnp.float32),
                        hi_ref[...])
        hi_ref[...] = jnp.maximum(hi0, part)

    @pl.when(ph == 1)
    def _():
        iou = iou_ref[:, pl.ds(t * T, T)]
        mv = jnp.max(iou, axis=0, keepdims=True)
        gidx = jax.lax.broadcasted_iota(jnp.int32, (G, T), 0)
        am = jnp.min(jnp.where(iou == mv, gidx, G), axis=0, keepdims=True)
        matches = jnp.where(mv < BG_THRESH, -1, am)
        matches = jnp.where((mv >= BG_THRESH) & (mv < FG_THRESH), -2,
                            matches)
        hi = hi_ref[...]
        ones_g = jnp.ones((1, G), jnp.float32)
        force = jax.lax.dot_general(
            ones_g, jnp.where(iou == hi, 1.0, 0.0),
            (((1,), (0,)), ((), ())),
            preferred_element_type=jnp.float32) > 0.5
        matches = jnp.where(force, am, matches)
        fgb = (matches >= 0) & padm
        validb = (matches != -2) & padm
        midx = jnp.maximum(matches, 0)
        onehot = (gidx == midx).astype(jnp.float32)

        g5 = jax.lax.dot_general(tb5_ref[0], onehot,
                                 (((1,), (0,)), ((), ())),
                                 preferred_element_type=jnp.float32)
        gx0, gy0 = g5[0:1, :], g5[1:2, :]
        gx1, gy1 = g5[2:3, :], g5[3:4, :]
        lab = g5[4:5, :].astype(jnp.int32)

        x = plabt_ref[0]
        e = jnp.exp(-jnp.abs(x))
        lg = jnp.log1p(e)
        r = 1.0 / (1.0 + e)
        xpos = x >= 0.0
        sp = jnp.where(xpos, x, 0.0) + lg
        p = jnp.where(xpos, r, e * r)
        negterm = sp * (p * p)
        ones_c = jnp.ones((1, C), jnp.float32)
        base = (1.0 - ALPHA) * jax.lax.dot_general(
            ones_c, negterm, (((1,), (0,)), ((), ())),
            preferred_element_type=jnp.float32)
        cidx = jax.lax.broadcasted_iota(jnp.int32, (C, T), 0)
        xsel = jax.lax.dot_general(
            ones_c, jnp.where(cidx == lab, x, 0.0),
            (((1,), (0,)), ((), ())),
            preferred_element_type=jnp.float32)
        e1 = jnp.exp(-jnp.abs(xsel))
        lg1 = jnp.log1p(e1)
        r1 = 1.0 / (1.0 + e1)
        xp1 = xsel >= 0.0
        sp1 = jnp.where(xp1, xsel, 0.0) + lg1
        sn1 = jnp.where(xp1, 0.0, -xsel) + lg1
        p1 = jnp.where(xp1, r1, e1 * r1)
        q1 = 1.0 - p1
        corr = ALPHA * sn1 * q1 * q1 - (1.0 - ALPHA) * sp1 * p1 * p1
        cls_sum = jnp.sum(jnp.where(validb, base, 0.0) +
                          jnp.where(fgb, corr, 0.0))

        anc = anct_ref[0]
        ax0, ay0 = anc[0:1, :], anc[1:2, :]
        ax1, ay1 = anc[2:3, :], anc[3:4, :]
        aw = ax1 - ax0
        ah = ay1 - ay0
        cx = ax0 + 0.5 * aw
        cy = ay0 + 0.5 * ah
        pbt = pbt_ref[0]
        dx, dy = pbt[0:1, :], pbt[1:2, :]
        dw = jnp.minimum(pbt[2:3, :], DW_CLAMP)
        dh = jnp.minimum(pbt[3:4, :], DW_CLAMP)
        pcx = dx * aw + cx
        pcy = dy * ah + cy
        pw = jnp.exp(dw) * aw
        phh0 = jnp.exp(dh) * ah
        px0 = pcx - 0.5 * pw
        py0 = pcy - 0.5 * phh0
        px1 = pcx + 0.5 * pw
        py1 = pcy + 0.5 * phh0

        iw = jnp.maximum(jnp.minimum(px1, gx1) - jnp.maximum(px0, gx0), 0.0)
        ih = jnp.maximum(jnp.minimum(py1, gy1) - jnp.maximum(py0, gy0), 0.0)
        binter = iw * ih
        pww = px1 - px0
        phh = py1 - py0
        gww = gx1 - gx0
        ghh = gy1 - gy0
        union = pww * phh + gww * ghh - binter
        biou = binter / (union + EPS)
        cw = jnp.maximum(px1, gx1) - jnp.minimum(px0, gx0)
        ch = jnp.maximum(py1, gy1) - jnp.minimum(py0, gy0)
        c2 = cw * cw + ch * ch + EPS
        rho2 = ((px0 + px1) * 0.5 - (gx0 + gx1) * 0.5) ** 2 + \
               ((py0 + py1) * 0.5 - (gy0 + gy1) * 0.5) ** 2
        v = (4.0 / (PI * PI)) * (_atan_pos(gww / (ghh + EPS)) -
                                 _atan_pos(pww / (phh + EPS))) ** 2
        alpha_t = v / (1.0 - biou + v + EPS)
        bl = 1.0 - (biou - rho2 / c2 - alpha_t * v)
        box_sum = jnp.sum(jnp.where(fgb, bl, 0.0))
        nfg = jnp.sum(jnp.where(fgb, 1.0, 0.0))

        lane = jax.lax.broadcasted_iota(jnp.int32, (1, 128), 1)
        row = jnp.where(lane == 0, cls_sum, 0.0) + \
              jnp.where(lane == 1, box_sum, 0.0) + \
              jnp.where(lane == 2, nfg, 0.0)

        @pl.when(t == 0)
        def _():
            acc_ref[0] = jnp.zeros((1, 128), jnp.float32)

        acc_ref[0] = acc_ref[0] + row


@jax.jit
def kernel(pred_boxes, pred_labels, target_boxes, target_labels, anchors):
    anct = jnp.transpose(anchors, (0, 2, 1))
    pbt = jnp.transpose(pred_boxes, (0, 2, 1))
    plabt = jnp.transpose(pred_labels, (0, 2, 1))
    tlf = target_labels.astype(jnp.float32)[:, :, None]
    tb5 = jnp.concatenate(
        [jnp.transpose(target_boxes, (0, 2, 1)), tlf.transpose(0, 2, 1)],
        axis=1)

    nt = pl.cdiv(N, T)
    acc = pl.pallas_call(
        _kernel,
        grid=(B, 2, nt),
        in_specs=[
            pl.BlockSpec((1, 4, T), lambda b, ph, t: (b, 0, t * ph)),
            pl.BlockSpec((1, C, T), lambda b, ph, t: (b, 0, t * ph)),
            pl.BlockSpec((1, 4, T), lambda b, ph, t: (b, 0, t)),
            pl.BlockSpec((1, G, 4), lambda b, ph, t: (b, 0, 0)),
            pl.BlockSpec((1, 5, G), lambda b, ph, t: (b, 0, 0)),
        ],
        out_specs=pl.BlockSpec((1, 1, 128), lambda b, ph, t: (b, 0, 0)),
        out_shape=jax.ShapeDtypeStruct((B, 1, 128), jnp.float32),
        scratch_shapes=[pltpu.VMEM((G, 1), jnp.float32),
                        pltpu.VMEM((G, T * ((N + T - 1) // T)), jnp.float32)],
    )(pbt, plabt, anct, target_boxes, tb5)

    cls_sum = acc[:, 0, 0]
    box_sum = acc[:, 0, 1]
    nfg = acc[:, 0, 2]
    denom = jnp.maximum(1.0, nfg)
    cls = cls_sum / denom
    box = box_sum / denom
    return jnp.stack([cls.mean(), box.mean()])


# speedup vs baseline: 2.6878x; 1.0138x over previous
import jax
import jax.numpy as jnp
import numpy as _np
from jax.experimental import pallas as pl
from jax.experimental.pallas import tpu as pltpu

B, N, G, C = 8, 20000, 64, 80
FG_THRESH, BG_THRESH = 0.5, 0.4
ALPHA, GAMMA = 0.25, 2.0
EPS = 1e-7
DW_CLAMP = float(_np.log(1000.0 / 16.0))
PI = float(_np.pi)

T = 5120


def _atan_pos(x):
    big = x > 1.0
    y = jnp.where(big, 1.0 / x, x)
    y = y / (1.0 + jnp.sqrt(1.0 + y * y))
    y = y / (1.0 + jnp.sqrt(1.0 + y * y))
    t2 = y * y
    at = y * (1.0 + t2 * (-1.0 / 3.0 + t2 * (1.0 / 5.0 + t2 * (-1.0 / 7.0 + t2 * (1.0 / 9.0)))))
    at = 4.0 * at
    return jnp.where(big, PI / 2.0 - at, at)


def _kernel(pbt_ref, plabt_ref, anct_ref, tbc_ref, tb5_ref, acc_ref,
            hi_ref, iou_ref):
    ph = pl.program_id(1)
    t = pl.program_id(2)

    col = t * T + jax.lax.broadcasted_iota(jnp.int32, (1, T), 1)
    padm = col < N

    @pl.when(ph == 0)
    def _():
        anc = anct_ref[0]
        tbc = tbc_ref[0]
        ax0, ay0 = anc[0:1, :], anc[1:2, :]
        ax1, ay1 = anc[2:3, :], anc[3:4, :]
        bx0, by0 = tbc[:, 0:1], tbc[:, 1:2]
        bx1, by1 = tbc[:, 2:3], tbc[:, 3:4]
        area_a = (ax1 - ax0) * (ay1 - ay0)
        area_b = (bx1 - bx0) * (by1 - by0)
        w = jnp.maximum(jnp.minimum(bx1, ax1) - jnp.maximum(bx0, ax0), 0.0)
        h = jnp.maximum(jnp.minimum(by1, ay1) - jnp.maximum(by0, ay0), 0.0)
        inter = w * h
        iou = inter / (area_b + area_a - inter + EPS)
        iou = jnp.where(padm, iou, -1.0)
        iou_ref[:, pl.ds(t * T, T)] = iou
        part = jnp.max(iou, axis=1, keepdims=True)
        hi0 = jnp.where(t == 0, jnp.full((G, 1), -jnp.inf, jnp.float32),
                        hi_ref[...])
        hi_ref[...] = jnp.maximum(hi0, part)

    @pl.when(ph == 1)
    def _():
        iou = iou_ref[:, pl.ds(t * T, T)]
        mv = jnp.max(iou, axis=0, keepdims=True)
        gidx = jax.lax.broadcasted_iota(jnp.int32, (G, T), 0)
        am = jnp.min(jnp.where(iou == mv, gidx, G), axis=0, keepdims=True)
        matches = jnp.where(mv < BG_THRESH, -1, am)
        matches = jnp.where((mv >= BG_THRESH) & (mv < FG_THRESH), -2,
                            matches)
        hi = hi_ref[...]
        ones_g = jnp.ones((1, G), jnp.float32)
        force = jax.lax.dot_general(
            ones_g, jnp.where(iou == hi, 1.0, 0.0),
            (((1,), (0,)), ((), ())),
            preferred_element_type=jnp.float32) > 0.5
        matches = jnp.where(force, am, matches)
        fgb = (matches >= 0) & padm
        validb = (matches != -2) & padm
        midx = jnp.maximum(matches, 0)
        onehot = (gidx == midx).astype(jnp.float32)

        g5 = jax.lax.dot_general(tb5_ref[0], onehot,
                                 (((1,), (0,)), ((), ())),
                                 preferred_element_type=jnp.float32)
        gx0, gy0 = g5[0:1, :], g5[1:2, :]
        gx1, gy1 = g5[2:3, :], g5[3:4, :]
        lab = g5[4:5, :].astype(jnp.int32)

        x = plabt_ref[0]
        e = jnp.exp(-jnp.abs(x))
        lg = jnp.log1p(e)
        r = 1.0 / (1.0 + e)
        xpos = x >= 0.0
        sp = jnp.where(xpos, x, 0.0) + lg
        p = jnp.where(xpos, r, e * r)
        negterm = sp * (p * p)
        ones_c = jnp.ones((1, C), jnp.float32)
        base = (1.0 - ALPHA) * jax.lax.dot_general(
            ones_c, negterm, (((1,), (0,)), ((), ())),
            preferred_element_type=jnp.float32)
        cidx = jax.lax.broadcasted_iota(jnp.int32, (C, T), 0)
        xsel = jax.lax.dot_general(
            ones_c, jnp.where(cidx == lab, x, 0.0),
            (((1,), (0,)), ((), ())),
            preferred_element_type=jnp.float32)
        e1 = jnp.exp(-jnp.abs(xsel))
        lg1 = jnp.log1p(e1)
        r1 = 1.0 / (1.0 + e1)
        xp1 = xsel >= 0.0
        sp1 = jnp.where(xp1, xsel, 0.0) + lg1
        sn1 = jnp.where(xp1, 0.0, -xsel) + lg1
        p1 = jnp.where(xp1, r1, e1 * r1)
        q1 = 1.0 - p1
        corr = ALPHA * sn1 * q1 * q1 - (1.0 - ALPHA) * sp1 * p1 * p1
        cls_sum = jnp.sum(jnp.where(validb, base, 0.0) +
                          jnp.where(fgb, corr, 0.0))

        anc = anct_ref[0]
        ax0, ay0 = anc[0:1, :], anc[1:2, :]
        ax1, ay1 = anc[2:3, :], anc[3:4, :]
        aw = ax1 - ax0
        ah = ay1 - ay0
        cx = ax0 + 0.5 * aw
        cy = ay0 + 0.5 * ah
        pbt = pbt_ref[0]
        dx, dy = pbt[0:1, :], pbt[1:2, :]
        dw = jnp.minimum(pbt[2:3, :], DW_CLAMP)
        dh = jnp.minimum(pbt[3:4, :], DW_CLAMP)
        pcx = dx * aw + cx
        pcy = dy * ah + cy
        pw = jnp.exp(dw) * aw
        phh0 = jnp.exp(dh) * ah
        px0 = pcx - 0.5 * pw
        py0 = pcy - 0.5 * phh0
        px1 = pcx + 0.5 * pw
        py1 = pcy + 0.5 * phh0

        iw = jnp.maximum(jnp.minimum(px1, gx1) - jnp.maximum(px0, gx0), 0.0)
        ih = jnp.maximum(jnp.minimum(py1, gy1) - jnp.maximum(py0, gy0), 0.0)
        binter = iw * ih
        pww = px1 - px0
        phh = py1 - py0
        gww = gx1 - gx0
        ghh = gy1 - gy0
        union = pww * phh + gww * ghh - binter
        biou = binter / (union + EPS)
        cw = jnp.maximum(px1, gx1) - jnp.minimum(px0, gx0)
        ch = jnp.maximum(py1, gy1) - jnp.minimum(py0, gy0)
        c2 = cw * cw + ch * ch + EPS
        rho2 = ((px0 + px1) * 0.5 - (gx0 + gx1) * 0.5) ** 2 + \
               ((py0 + py1) * 0.5 - (gy0 + gy1) * 0.5) ** 2
        v = (4.0 / (PI * PI)) * (_atan_pos(gww / (ghh + EPS)) -
                                 _atan_pos(pww / (phh + EPS))) ** 2
        alpha_t = v / (1.0 - biou + v + EPS)
        bl = 1.0 - (biou - rho2 / c2 - alpha_t * v)
        box_sum = jnp.sum(jnp.where(fgb, bl, 0.0))
        nfg = jnp.sum(jnp.where(fgb, 1.0, 0.0))

        lane = jax.lax.broadcasted_iota(jnp.int32, (1, 128), 1)
        row = jnp.where(lane == 0, cls_sum, 0.0) + \
              jnp.where(lane == 1, box_sum, 0.0) + \
              jnp.where(lane == 2, nfg, 0.0)

        @pl.when(t == 0)
        def _():
            acc_ref[0] = jnp.zeros((1, 128), jnp.float32)

        acc_ref[0] = acc_ref[0] + row


@jax.jit
def kernel(pred_boxes, pred_labels, target_boxes, target_labels, anchors):
    anct = jnp.transpose(anchors, (0, 2, 1))
    pbt = jnp.transpose(pred_boxes, (0, 2, 1))
    plabt = jnp.transpose(pred_labels, (0, 2, 1))
    tlf = target_labels.astype(jnp.float32)[:, :, None]
    tb5 = jnp.concatenate(
        [jnp.transpose(target_boxes, (0, 2, 1)), tlf.transpose(0, 2, 1)],
        axis=1)

    nt = pl.cdiv(N, T)
    acc = pl.pallas_call(
        _kernel,
        grid=(B, 2, nt),
        in_specs=[
            pl.BlockSpec((1, 4, T), lambda b, ph, t: (b, 0, t * ph)),
            pl.BlockSpec((1, C, T), lambda b, ph, t: (b, 0, t * ph)),
            pl.BlockSpec((1, 4, T), lambda b, ph, t: (b, 0, t)),
            pl.BlockSpec((1, G, 4), lambda b, ph, t: (b, 0, 0)),
            pl.BlockSpec((1, 5, G), lambda b, ph, t: (b, 0, 0)),
        ],
        out_specs=pl.BlockSpec((1, 1, 128), lambda b, ph, t: (b, 0, 0)),
        out_shape=jax.ShapeDtypeStruct((B, 1, 128), jnp.float32),
        scratch_shapes=[pltpu.VMEM((G, 1), jnp.float32),
                        pltpu.VMEM((G, T * ((N + T - 1) // T)), jnp.float32)],
    )(pbt, plabt, anct, target_boxes, tb5)

    cls_sum = acc[:, 0, 0]
    box_sum = acc[:, 0, 1]
    nfg = acc[:, 0, 2]
    denom = jnp.maximum(1.0, nfg)
    cls = cls_sum / denom
    box = box_sum / denom
    return jnp.stack([cls.mean(), box.mean()])
